# Initial kernel scaffold; baseline (speedup 1.0000x reference)
import jax
import jax.numpy as jnp
from jax.experimental import pallas as pl


def kernel(x, edge_index, edge_attr, params):
    raise NotImplementedError("write your pallas kernel here")


# speedup vs baseline: 5.1743x; 5.1743x over previous
import functools

import jax
import jax.numpy as jnp
from jax import lax
from jax.experimental import pallas as pl
from jax.experimental.pallas import tpu as pltpu
from jax.experimental.pallas import tpu_sc as plsc

N = 4096
E = 262144
IN_DIM = 128
DM = 16
TP = 4224
NTOK = N + 1
QBLK = 128
NQB = TP // QBLK
_BN_SCALE = 1.0 / (1.0 + 1e-5) ** 0.5

f32 = jnp.float32


def _ln(x, g, b):
    mu = jnp.mean(x, axis=-1, keepdims=True)
    xc = x - mu
    var = jnp.mean(xc * xc, axis=-1, keepdims=True)
    return xc * lax.rsqrt(var + 1e-5) * g + b


def _softmax(s):
    m = jnp.max(s, axis=-1, keepdims=True)
    e = jnp.exp(s - m)
    return e / jnp.sum(e, axis=-1, keepdims=True)


def _dot(a, b):
    return jnp.dot(a, b, preferred_element_type=f32)


def _node_pre_body(x_ref, bng, bnb, w1x, embw, embb, u_ref, t0_ref):
    xb = x_ref[...] * (bng[...] * _BN_SCALE) + bnb[...]
    u_ref[...] = _dot(xb, w1x[...])
    t0_ref[...] = _dot(xb, embw[...]) + embb[...]


def _node_pre(x, bng, bnb, w1x, embw, embb):
    return pl.pallas_call(
        _node_pre_body,
        out_shape=[
            jax.ShapeDtypeStruct((N, DM), f32),
            jax.ShapeDtypeStruct((N, DM), f32),
        ],
    )(x, bng, bnb, w1x, embw, embb)


_BE = 8192


def _enc_mlp_body(g1, ea, w1e, b1, w2, b2, w3, b3, w4, b4, out):
    a = _dot(ea[...], w1e[...]) + g1[...] + b1[...]
    a = jnp.maximum(a, 0.0)
    a = jnp.maximum(_dot(a, w2[...]) + b2[...], 0.0)
    a = jnp.maximum(_dot(a, w3[...]) + b3[...], 0.0)
    out[...] = jnp.maximum(_dot(a, w4[...]) + b4[...], 0.0)


def _full(shape):
    return pl.BlockSpec(shape, lambda i: (0, 0))


def _row_spec():
    return pl.BlockSpec((_BE, DM), lambda i: (i, 0))


def _enc_mlp(g1, ea, w1e, b1, w2, b2, w3, b3, w4, b4):
    return pl.pallas_call(
        _enc_mlp_body,
        grid=(E // _BE,),
        in_specs=[
            _row_spec(),
            pl.BlockSpec((_BE, 16), lambda i: (i, 0)),
            _full((16, 16)), _full((1, 16)),
            _full((16, 32)), _full((1, 32)),
            _full((32, 16)), _full((1, 16)),
            _full((16, 16)), _full((1, 16)),
        ],
        out_specs=_row_spec(),
        out_shape=jax.ShapeDtypeStruct((E, DM), f32),
    )(g1, ea, w1e, b1, w2, b2, w3, b3, w4, b4)


def _dec_mlp_body(gd, b1, w2, b2, w3, b3, out):
    a = jnp.maximum(gd[...] + b1[...], 0.0)
    a = jnp.maximum(_dot(a, w2[...]) + b2[...], 0.0)
    out[...] = jnp.maximum(_dot(a, w3[...]) + b3[...], 0.0)


def _dec_mlp(gd, b1, w2, b2, w3, b3):
    return pl.pallas_call(
        _dec_mlp_body,
        grid=(E // _BE,),
        in_specs=[
            _row_spec(),
            _full((1, 16)),
            _full((16, 32)), _full((1, 32)),
            _full((32, 16)), _full((1, 16)),
        ],
        out_specs=_row_spec(),
        out_shape=jax.ShapeDtypeStruct((E, DM), f32),
    )(gd, b1, w2, b2, w3, b3)


def _attn(qb, k, v, maskneg):
    inv = 1.0 / (8.0 ** 0.5)
    outs = []
    for h0 in (0, 8):
        s = lax.dot_general(qb[:, h0:h0 + 8], k[:, h0:h0 + 8],
                            (((1,), (1,)), ((), ()))) * inv
        p = _softmax(s + maskneg)
        outs.append(_dot(p, v[:, h0:h0 + 8]))
    return jnp.concatenate(outs, axis=1)


def _trans_body(*args):
    tin = args[0]
    W = args[1:33]
    g_ref = args[33]
    t_s, k_s, v_s = args[34:37]
    col = lax.broadcasted_iota(jnp.int32, (1, TP), 1)
    maskneg = jnp.where(col < NTOK, 0.0, -1e30)

    t_s[...] = tin[...]

    (wq, bq, wk, bk, wv, bv, wo, bo,
     fw1, fb1, fw2, fb2, l1g, l1b, l2g, l2b) = W[0:16]
    tcur = t_s[...]
    k_s[...] = _dot(tcur, wk[...]) + bk[...]
    v_s[...] = _dot(tcur, wv[...]) + bv[...]

    def blk(i, carry):
        r0 = pl.multiple_of(i * QBLK, QBLK)
        tb = t_s[pl.ds(r0, QBLK), :]
        qb = _dot(tb, wq[...]) + bq[...]
        a = _attn(qb, k_s[...], v_s[...], maskneg)
        a = _dot(a, wo[...]) + bo[...]
        x1 = _ln(tb + a, l1g[...], l1b[...])
        ff = _dot(jnp.maximum(_dot(x1, fw1[...]) + fb1[...], 0.0),
                  fw2[...]) + fb2[...]
        t_s[pl.ds(r0, QBLK), :] = _ln(x1 + ff, l2g[...], l2b[...])
        return carry

    lax.fori_loop(0, NQB, blk, 0)

    (wq, bq, wk, bk, wv, bv, wo, bo,
     fw1, fb1, fw2, fb2, l1g, l1b, l2g, l2b) = W[16:32]
    tcur = t_s[...]
    k2 = _dot(tcur, wk[...]) + bk[...]
    v2 = _dot(tcur, wv[...]) + bv[...]
    tb = t_s[pl.ds(N, 8), :]
    qb = _dot(tb, wq[...]) + bq[...]
    a = _attn(qb, k2, v2, maskneg)
    a = _dot(a, wo[...]) + bo[...]
    x1 = _ln(tb + a, l1g[...], l1b[...])
    ff = _dot(jnp.maximum(_dot(x1, fw1[...]) + fb1[...], 0.0),
              fw2[...]) + fb2[...]
    g_ref[...] = _ln(x1 + ff, l2g[...], l2b[...])


def _transformer(tpad, wlist):
    return pl.pallas_call(
        _trans_body,
        out_shape=jax.ShapeDtypeStruct((8, DM), f32),
        scratch_shapes=[pltpu.VMEM((TP, DM), f32)] * 3,
    )(tpad, *wlist)


def _ca_body(zs, cn, g8, projw, projb, caq, cabq, cak, cabk, cav, cabv,
             cao, cabo, c2nw, c2nb, dw1, hA_ref, hB_ref):
    cnt = cn[0:N, 0:1] + cn[N:2 * N, 0:1]
    z = (zs[0:N, :] + zs[N:2 * N, :]) / jnp.maximum(cnt, 1.0)
    zp = _dot(z, projw[...]) + projb[...]
    q = _dot(g8[...], caq[...]) + cabq[...]
    k = _dot(zp, cak[...]) + cabk[...]
    v = _dot(zp, cav[...]) + cabv[...]
    s = lax.dot_general(q, k, (((1,), (1,)), ((), ()))) * 0.25
    p = _softmax(s)
    att = _dot(p, v)
    o = _dot(att, cao[...]) + cabo[...]
    refined = _dot(o, c2nw[...]) + c2nb[...]
    h = zp + refined[0:1, :]
    wa = dw1[0:16, :] - dw1[16:32, :]
    wb = dw1[16:32, :]
    hA_ref[...] = _dot(h, wa)
    hB_ref[...] = _dot(h, wb)


def _ca(zs, cn, g8, projw, projb, caq, cabq, cak, cabk, cav, cabv,
        cao, cabo, c2nw, c2nb, dw1):
    return pl.pallas_call(
        _ca_body,
        out_shape=[
            jax.ShapeDtypeStruct((N, DM), f32),
            jax.ShapeDtypeStruct((N, DM), f32),
        ],
    )(zs, cn, g8, projw, projb, caq, cabq, cak, cabk, cav, cabv,
      cao, cabo, c2nw, c2nb, dw1)


def _final_body(ms, cn, w4, b4, out_ref):
    cnt = cn[0:N, 0:1] + cn[N:2 * N, 0:1]
    m3 = (ms[0:N, :] + ms[N:2 * N, :]) / jnp.maximum(cnt, 1.0)
    out_ref[...] = _dot(m3, w4[...]) + b4[...] * jnp.minimum(cnt, 1.0)


def _final(ms, cn, w4, b4):
    return pl.pallas_call(
        _final_body,
        out_shape=jax.ShapeDtypeStruct((N, IN_DIM), f32),
    )(ms, cn, w4, b4)


_NW = 32
_CPT = (E // 128) // _NW


def _sc_mesh():
    return plsc.VectorSubcoreMesh(core_axis_name="c", subcore_axis_name="s")


def _enc_gather(u, dst2d):
    @functools.partial(
        pl.kernel,
        out_type=jax.ShapeDtypeStruct((E, DM), f32),
        mesh=_sc_mesh(),
        compiler_params=pltpu.CompilerParams(use_tc_tiling_on_sc=False),
        scratch_types=[
            pltpu.VMEM((_CPT, 128), jnp.int32),
            pltpu.VMEM((2048, DM), f32),
            pltpu.SemaphoreType.DMA,
        ],
    )
    def kfn(u_hbm, dst_hbm, out_hbm, idx_v, rows_v, sem):
        w = lax.axis_index("s") * 2 + lax.axis_index("c")
        c0 = w * _CPT
        pltpu.sync_copy(dst_hbm.at[pl.ds(c0, _CPT)], idx_v)

        def outer(o, carry):
            descs = []
            for j in range(16):
                descs.append(pltpu.async_copy(
                    u_hbm.at[idx_v.at[o * 16 + j]],
                    rows_v.at[pl.ds(j * 128, 128)], sem))
            for d in descs:
                d.wait()
            pltpu.sync_copy(rows_v,
                            out_hbm.at[pl.ds((c0 + o * 16) * 128, 2048)])
            return carry

        lax.fori_loop(0, _CPT // 16, outer, 0)

    return kfn(u, dst2d)


def _dec_gather(hA, hB, dst2d, src2d):
    @functools.partial(
        pl.kernel,
        out_type=jax.ShapeDtypeStruct((E, DM), f32),
        mesh=_sc_mesh(),
        compiler_params=pltpu.CompilerParams(use_tc_tiling_on_sc=False),
        scratch_types=[
            pltpu.VMEM((_CPT, 128), jnp.int32),
            pltpu.VMEM((_CPT, 128), jnp.int32),
            pltpu.VMEM((1024, DM), f32),
            pltpu.VMEM((1024, DM), f32),
            pltpu.SemaphoreType.DMA,
            pltpu.SemaphoreType.DMA,
        ],
    )
    def kfn(hA_hbm, hB_hbm, dst_hbm, src_hbm, out_hbm,
            idxd, idxs, bufA, bufB, semA, semB):
        w = lax.axis_index("s") * 2 + lax.axis_index("c")
        c0 = w * _CPT
        pltpu.sync_copy(dst_hbm.at[pl.ds(c0, _CPT)], idxd)
        pltpu.sync_copy(src_hbm.at[pl.ds(c0, _CPT)], idxs)

        def outer(o, carry):
            descs = []
            for j in range(8):
                descs.append(pltpu.async_copy(
                    hA_hbm.at[idxd.at[o * 8 + j]],
                    bufA.at[pl.ds(j * 128, 128)], semA))
                descs.append(pltpu.async_copy(
                    hB_hbm.at[idxs.at[o * 8 + j]],
                    bufB.at[pl.ds(j * 128, 128)], semB))
            for d in descs:
                d.wait()

            def addrow(r, c):
                bufA[r, :] = bufA[r, :] + bufB[r, :]
                return c

            lax.fori_loop(0, 1024, addrow, 0)
            pltpu.sync_copy(bufA,
                            out_hbm.at[pl.ds((c0 + o * 8) * 128, 1024)])
            return carry

        lax.fori_loop(0, _CPT // 8, outer, 0)

    return kfn(hA, hB, dst2d, src2d)


def _enc_scatter(m4, dst2d):
    @functools.partial(
        pl.kernel,
        out_type=[
            jax.ShapeDtypeStruct((2 * N, DM), f32),
            jax.ShapeDtypeStruct((2 * N, DM), f32),
        ],
        mesh=_sc_mesh(),
        compiler_params=pltpu.CompilerParams(use_tc_tiling_on_sc=False),
        scratch_types=[
            pltpu.VMEM((_CPT, 128), jnp.int32),
            pltpu.VMEM((1024, DM), f32),
            pltpu.VMEM((128, DM), f32),
            pltpu.VMEM((256, DM), f32),
            pltpu.VMEM_SHARED((N, DM), f32),
            pltpu.VMEM_SHARED((N, DM), f32),
        ],
    )
    def kfn(m4_hbm, dst_hbm, zout, cout, idx_v, dbuf, ones_v, zbuf,
            z_acc, c_acc):
        cid = lax.axis_index("c")
        sid = lax.axis_index("s")
        w = sid * 2 + cid
        c0 = w * _CPT

        def zrow(r, c):
            zbuf[r, :] = jnp.zeros((DM,), f32)
            return c

        lax.fori_loop(0, 256, zrow, 0)

        def onerow(r, c):
            ones_v[r, :] = jnp.ones((DM,), f32)
            return c

        lax.fori_loop(0, 128, onerow, 0)
        pltpu.sync_copy(zbuf, z_acc.at[pl.ds(sid * 256, 256)])
        pltpu.sync_copy(zbuf, c_acc.at[pl.ds(sid * 256, 256)])
        plsc.subcore_barrier()

        pltpu.sync_copy(dst_hbm.at[pl.ds(c0, _CPT)], idx_v)

        def outer(o, carry):
            pltpu.sync_copy(m4_hbm.at[pl.ds((c0 + o * 8) * 128, 1024)], dbuf)
            for j in range(8):
                pltpu.sync_copy(dbuf.at[pl.ds(j * 128, 128)],
                                z_acc.at[idx_v.at[o * 8 + j]], add=True)
                pltpu.sync_copy(ones_v,
                                c_acc.at[idx_v.at[o * 8 + j]], add=True)
            return carry

        lax.fori_loop(0, _CPT // 8, outer, 0)
        plsc.subcore_barrier()
        pltpu.sync_copy(z_acc.at[pl.ds(sid * 256, 256)],
                        zout.at[pl.ds(cid * N + sid * 256, 256)])
        pltpu.sync_copy(c_acc.at[pl.ds(sid * 256, 256)],
                        cout.at[pl.ds(cid * N + sid * 256, 256)])

    return kfn(m4, dst2d)


def _dec_scatter(d3, dst2d):
    @functools.partial(
        pl.kernel,
        out_type=jax.ShapeDtypeStruct((2 * N, DM), f32),
        mesh=_sc_mesh(),
        compiler_params=pltpu.CompilerParams(use_tc_tiling_on_sc=False),
        scratch_types=[
            pltpu.VMEM((_CPT, 128), jnp.int32),
            pltpu.VMEM((1024, DM), f32),
            pltpu.VMEM((256, DM), f32),
            pltpu.VMEM_SHARED((N, DM), f32),
        ],
    )
    def kfn(d3_hbm, dst_hbm, mout, idx_v, dbuf, zbuf, m_acc):
        cid = lax.axis_index("c")
        sid = lax.axis_index("s")
        w = sid * 2 + cid
        c0 = w * _CPT

        def zrow(r, c):
            zbuf[r, :] = jnp.zeros((DM,), f32)
            return c

        lax.fori_loop(0, 256, zrow, 0)
        pltpu.sync_copy(zbuf, m_acc.at[pl.ds(sid * 256, 256)])
        plsc.subcore_barrier()

        pltpu.sync_copy(dst_hbm.at[pl.ds(c0, _CPT)], idx_v)

        def outer(o, carry):
            pltpu.sync_copy(d3_hbm.at[pl.ds((c0 + o * 8) * 128, 1024)], dbuf)
            for j in range(8):
                pltpu.sync_copy(dbuf.at[pl.ds(j * 128, 128)],
                                m_acc.at[idx_v.at[o * 8 + j]], add=True)
            return carry

        lax.fori_loop(0, _CPT // 8, outer, 0)
        plsc.subcore_barrier()
        pltpu.sync_copy(m_acc.at[pl.ds(sid * 256, 256)],
                        mout.at[pl.ds(cid * N + sid * 256, 256)])

    return kfn(d3, dst2d)


def kernel(x, edge_index, edge_attr, params):
    p = params
    r2 = lambda a: a.reshape(1, -1)
    src2d = edge_index[0].reshape(E // 128, 128)
    dst2d = edge_index[1].reshape(E // 128, 128)

    u, t0 = _node_pre(x, r2(p['bn_g']), r2(p['bn_b']),
                      p['enc_W1'][:IN_DIM], p['emb_W'], r2(p['emb_b']))
    g1 = _enc_gather(u, dst2d)
    m4 = _enc_mlp(g1, edge_attr,
                  p['enc_W1'][IN_DIM:], r2(p['enc_b1']),
                  p['enc_W2'], r2(p['enc_b2']),
                  p['enc_W3'], r2(p['enc_b3']),
                  p['enc_W4'], r2(p['enc_b4']))
    zsum, cnt = _enc_scatter(m4, dst2d)

    tpad = jnp.concatenate(
        [t0, p['cls'][None, :], jnp.zeros((TP - N - 1, DM), f32)], axis=0)
    wlist = []
    for li in range(2):
        pre = 'tl%d_' % li
        for nm in ('Wq', 'bq', 'Wk', 'bk', 'Wv', 'bv', 'Wo', 'bo',
                   'fW1', 'fb1', 'fW2', 'fb2',
                   'ln1g', 'ln1b', 'ln2g', 'ln2b'):
            a = p[pre + nm]
            wlist.append(a if a.ndim == 2 else r2(a))
    g8 = _transformer(tpad, wlist)

    hA, hB = _ca(zsum, cnt, g8,
                 p['proj_W'], r2(p['proj_b']),
                 p['ca_Wq'], r2(p['ca_bq']),
                 p['ca_Wk'], r2(p['ca_bk']),
                 p['ca_Wv'], r2(p['ca_bv']),
                 p['ca_Wo'], r2(p['ca_bo']),
                 p['c2n_W'], r2(p['c2n_b']),
                 p['dec_W1'])
    gd = _dec_gather(hA, hB, dst2d, src2d)
    d3 = _dec_mlp(gd, r2(p['dec_b1']),
                  p['dec_W2'], r2(p['dec_b2']),
                  p['dec_W3'], r2(p['dec_b3']))
    m3 = _dec_scatter(d3, dst2d)
    return _final(m3, cnt, p['dec_W4'], r2(p['dec_b4']))
